# Initial kernel scaffold; baseline (speedup 1.0000x reference)
import functools

import jax
import jax.numpy as jnp
from jax import lax
from jax.experimental import pallas as pl
from jax.experimental.pallas import tpu as pltpu
from jax.experimental.pallas import tpu_sc as plsc

D = 128
L = 16
NCHUNK = D // L


def _rsqrt_newton(v):
    i = lax.bitcast_convert_type(v, jnp.int32)
    i = jnp.int32(0x5F3759DF) - lax.shift_right_arithmetic(i, 1)
    y = lax.bitcast_convert_type(i, jnp.float32)
    half_v = 0.5 * v
    for _ in range(4):
        y = y * (1.5 - half_v * y * y)
    return y


def _make_kernel(n_rows, seq_len, ch):
    info = plsc.get_sparse_core_info()
    nw = info.num_cores * info.num_subcores
    rows_per_w = n_rows // nw
    n_chunks = rows_per_w // ch
    s_chunks = seq_len // ch

    mesh = plsc.VectorSubcoreMesh(core_axis_name="c", subcore_axis_name="s")

    @functools.partial(
        pl.kernel,
        out_type=jax.ShapeDtypeStruct((n_rows, D), jnp.float32),
        mesh=mesh,
        scratch_types=[
            pltpu.VMEM((seq_len, D), jnp.float32),
            pltpu.VMEM((3, D), jnp.float32),
            pltpu.VMEM((2, D), jnp.float32),
            pltpu.VMEM((ch,), jnp.int32),
            pltpu.VMEM((ch,), jnp.int32),
            pltpu.VMEM((ch, D), jnp.float32),
            pltpu.SemaphoreType.DMA,
        ],
    )
    def k(ids_hbm, segids_hbm, table_hbm, seg_hbm, pos_hbm, gamma_hbm,
          beta_hbm, out_hbm, pos_v, seg_v, gb_v, idx_v, sid_v, rows_v, sem):
        wid = lax.axis_index("s") * info.num_cores + lax.axis_index("c")
        base0 = wid * rows_per_w

        pltpu.sync_copy(pos_hbm, pos_v)
        pltpu.sync_copy(seg_hbm, seg_v)
        pltpu.sync_copy(gamma_hbm, gb_v.at[0])
        pltpu.sync_copy(beta_hbm, gb_v.at[1])

        seg_r = [[seg_v[r, pl.ds(c * L, L)] for c in range(NCHUNK)]
                 for r in range(3)]
        gam_r = [gb_v[0, pl.ds(c * L, L)] for c in range(NCHUNK)]
        bet_r = [gb_v[1, pl.ds(c * L, L)] for c in range(NCHUNK)]

        def chunk_body(g, carry):
            base = base0 + g * ch
            pltpu.sync_copy(ids_hbm.at[pl.ds(base, ch)], idx_v)
            pltpu.sync_copy(segids_hbm.at[pl.ds(base, ch)], sid_v)
            pltpu.async_copy(table_hbm.at[idx_v], rows_v, sem).wait()
            s0 = lax.rem(g, s_chunks) * ch

            def row_body(j, carry2):
                segv = plsc.load_gather(sid_v, [jnp.full((L,), j, jnp.int32)])
                m1 = segv >= 1
                m2 = segv >= 2
                x = []
                acc = None
                acc2 = None
                for c in range(NCHUNK):
                    t = rows_v[j, pl.ds(c * L, L)]
                    p = pos_v[s0 + j, pl.ds(c * L, L)]
                    sg = jnp.where(m2, seg_r[2][c],
                                   jnp.where(m1, seg_r[1][c], seg_r[0][c]))
                    xc = t + p + sg
                    x.append(xc)
                    acc = xc if acc is None else acc + xc
                    acc2 = xc * xc if acc2 is None else acc2 + xc * xc
                mean = jnp.sum(acc) * (1.0 / D)
                msq = jnp.sum(acc2) * (1.0 / D)
                var = msq - mean * mean
                r = _rsqrt_newton(var + 1e-12)
                for c in range(NCHUNK):
                    y = (x[c] - mean) * r * gam_r[c] + bet_r[c]
                    rows_v[j, pl.ds(c * L, L)] = y
                return carry2

            lax.fori_loop(0, ch, row_body, 0)
            pltpu.sync_copy(rows_v, out_hbm.at[pl.ds(base, ch)])
            return carry

        lax.fori_loop(0, n_chunks, chunk_body, 0)

    return k


def kernel(input_ids, segment_ids, token_table, segment_table, pos_emb,
           gamma, beta):
    b, s = input_ids.shape
    n_rows = b * s
    ids = input_ids.reshape(n_rows)
    sids = segment_ids.reshape(n_rows)
    k = _make_kernel(n_rows, s, 128)
    out = k(ids, sids, token_table, segment_table, pos_emb, gamma, beta)
    return out.reshape(b, s, D)


# speedup vs baseline: 2.3534x; 2.3534x over previous
import functools

import jax
import jax.numpy as jnp
from jax import lax
from jax.experimental import pallas as pl
from jax.experimental.pallas import tpu as pltpu
from jax.experimental.pallas import tpu_sc as plsc

D = 128
L = 16
NCHUNK = D // L


def _rsqrt_newton(v):
    i = lax.bitcast_convert_type(v, jnp.int32)
    i = jnp.int32(0x5F3759DF) - lax.shift_right_arithmetic(i, 1)
    y = lax.bitcast_convert_type(i, jnp.float32)
    half_v = 0.5 * v
    for _ in range(4):
        y = y * (1.5 - half_v * y * y)
    return y


def _make_kernel(n_rows, seq_len, ch):
    info = plsc.get_sparse_core_info()
    nw = info.num_cores * info.num_subcores
    rows_per_w = n_rows // nw
    n_chunks = rows_per_w // ch
    s_chunks = seq_len // ch

    mesh = plsc.VectorSubcoreMesh(core_axis_name="c", subcore_axis_name="s")

    @functools.partial(
        pl.kernel,
        out_type=jax.ShapeDtypeStruct((n_rows, D), jnp.float32),
        mesh=mesh,
        compiler_params=pltpu.CompilerParams(needs_layout_passes=False),
        scratch_types=[
            pltpu.VMEM((seq_len, D), jnp.float32),
            pltpu.VMEM((3, D), jnp.float32),
            pltpu.VMEM((2, D), jnp.float32),
            pltpu.VMEM((ch,), jnp.int32),
            pltpu.VMEM((ch + L,), jnp.int32),
            pltpu.VMEM((ch, D), jnp.float32),
            pltpu.SemaphoreType.DMA,
        ],
    )
    def k(ids_hbm, segids_hbm, table_hbm, seg_hbm, pos_hbm, gamma_hbm,
          beta_hbm, out_hbm, pos_v, seg_v, gb_v, idx_v, sid_v, rows_v, sem):
        wid = lax.axis_index("s") * info.num_cores + lax.axis_index("c")
        base0 = wid * rows_per_w

        pltpu.sync_copy(pos_hbm, pos_v)
        pltpu.sync_copy(seg_hbm, seg_v)
        pltpu.sync_copy(gamma_hbm, gb_v.at[0])
        pltpu.sync_copy(beta_hbm, gb_v.at[1])

        seg0_r = [seg_v[0, pl.ds(c * L, L)] for c in range(NCHUNK)]
        d1_r = [seg_v[1, pl.ds(c * L, L)] - seg0_r[c] for c in range(NCHUNK)]
        d2_r = [seg_v[2, pl.ds(c * L, L)] - seg_v[1, pl.ds(c * L, L)]
                for c in range(NCHUNK)]
        gam_r = [gb_v[0, pl.ds(c * L, L)] for c in range(NCHUNK)]
        bet_r = [gb_v[1, pl.ds(c * L, L)] for c in range(NCHUNK)]

        def chunk_body(g, carry):
            base = base0 + g * ch
            pltpu.sync_copy(ids_hbm.at[pl.ds(base, ch)], idx_v)
            pltpu.sync_copy(segids_hbm.at[pl.ds(base, ch)],
                            sid_v.at[pl.ds(0, ch)])
            pltpu.async_copy(table_hbm.at[idx_v], rows_v, sem).wait()
            s0 = lax.rem(g, s_chunks) * ch

            def row_body(j, carry2):
                sid = sid_v[pl.ds(j, L)][0]
                w1 = (sid >= 1).astype(jnp.float32)
                w2 = (sid >= 2).astype(jnp.float32)
                x = []
                acc = None
                acc2 = None
                for c in range(NCHUNK):
                    t = rows_v[j, pl.ds(c * L, L)]
                    p = pos_v[s0 + j, pl.ds(c * L, L)]
                    sg = seg0_r[c] + w1 * d1_r[c] + w2 * d2_r[c]
                    xc = t + p + sg
                    x.append(xc)
                    acc = xc if acc is None else acc + xc
                    acc2 = xc * xc if acc2 is None else acc2 + xc * xc
                mean = jnp.sum(acc) * (1.0 / D)
                msq = jnp.sum(acc2) * (1.0 / D)
                var = msq - mean * mean
                r = _rsqrt_newton(var + 1e-12)
                for c in range(NCHUNK):
                    y = (x[c] - mean) * r * gam_r[c] + bet_r[c]
                    rows_v[j, pl.ds(c * L, L)] = y
                return carry2

            lax.fori_loop(0, ch, row_body, 0)
            pltpu.sync_copy(rows_v, out_hbm.at[pl.ds(base, ch)])
            return carry

        lax.fori_loop(0, n_chunks, chunk_body, 0)

    return k


def kernel(input_ids, segment_ids, token_table, segment_table, pos_emb,
           gamma, beta):
    b, s = input_ids.shape
    n_rows = b * s
    ids = input_ids.reshape(n_rows)
    sids = segment_ids.reshape(n_rows)
    k = _make_kernel(n_rows, s, 128)
    out = k(ids, sids, token_table, segment_table, pos_emb, gamma, beta)
    return out.reshape(b, s, D)
